# baseline (speedup 1.0000x reference)
import functools

import jax
import jax.numpy as jnp
from jax import lax
from jax.experimental import pallas as pl
from jax.experimental.pallas import tpu as pltpu
from jax.experimental.pallas import tpu_sc as plsc

N = 10000
NPAD = 10240
E = 160000
NB = 40
DB = 128
BL = 120
EPAD = 16 * 88 * BL
EPADD = 2 * 16 * NB * DB
RPT = NPAD // 16
RBLK = 1024
NBUF = 2
NBT = 88
NB0 = 64
NB1 = NBT - NB0


def _mesh():
    return plsc.VectorSubcoreMesh(core_axis_name="c", subcore_axis_name="s")


@functools.partial(
    pl.kernel,
    out_type=jax.ShapeDtypeStruct((2, NPAD, 128), jnp.float32),
    mesh=_mesh(),
    name="degk",
    scratch_types=[
        pltpu.VMEM((NB, DB), jnp.int32),
        pltpu.VMEM((DB, 128), jnp.float32),
        pltpu.VMEM((64, 128), jnp.float32),
        pltpu.VMEM_SHARED((NPAD, 128), jnp.float32),
    ],
)
def _deg_kernel(dst_hbm, ones_hbm, zeros_hbm, out_hbm, dst_v, ones_v, zeros_v, acc):
    cid = lax.axis_index("c")
    sid = lax.axis_index("s")
    base = sid * RPT
    pltpu.sync_copy(dst_hbm.at[cid, sid], dst_v)
    pltpu.sync_copy(ones_hbm, ones_v)
    pltpu.sync_copy(zeros_hbm, zeros_v)
    for z in range(RPT // 64):
        pltpu.sync_copy(zeros_v, acc.at[pl.ds(base + z * 64, 64)])
    if RPT % 64:
        pltpu.sync_copy(zeros_v.at[pl.ds(0, RPT % 64)],
                        acc.at[pl.ds(base + (RPT // 64) * 64, RPT % 64)])
    plsc.subcore_barrier()

    def body(b, carry):
        pltpu.sync_copy(ones_v, acc.at[dst_v.at[b]], add=True)
        return carry

    lax.fori_loop(0, NB, body, 0)
    plsc.subcore_barrier()
    pltpu.sync_copy(acc.at[pl.ds(base, RPT)], out_hbm.at[cid, pl.ds(base, RPT)])


def _make_agg(nchunk):
    @functools.partial(
        pl.kernel,
        out_type=jax.ShapeDtypeStruct((2, nchunk, NPAD, 128), jnp.float32),
        mesh=_mesh(),
        name="agg%d" % nchunk,
        scratch_types=[
            pltpu.VMEM((NB0, BL), jnp.int32),
            pltpu.VMEM((NB0, BL), jnp.int32),
            [pltpu.VMEM((BL, 128), jnp.float32)] * NBUF,
            pltpu.VMEM_SHARED((NPAD, 128), jnp.float32),
            [pltpu.SemaphoreType.DMA] * NBUF,
            [pltpu.SemaphoreType.DMA] * NBUF,
        ],
    )
    def _agg(g_hbm, src_hbm, dst_hbm, zeros_hbm, out_hbm,
             src_v, dst_v, bufs, acc, gsems, ssems):
        cid = lax.axis_index("c")
        sid = lax.axis_index("s")
        base = sid * RPT
        nw = lax.select(cid == 0, NB0 // NBUF, NB1 // NBUF)

        def start_g(b, k):
            pltpu.async_copy(g_hbm.at[src_v.at[b]], bufs[k], gsems[k])

        def wait_g(b, k):
            pltpu.make_async_copy(g_hbm.at[src_v.at[b]], bufs[k], gsems[k]).wait()

        def start_s(b, k):
            pltpu.async_copy(bufs[k], acc.at[dst_v.at[b]], ssems[k], add=True)

        def wait_s(b, k):
            pltpu.make_async_copy(bufs[k], acc.at[dst_v.at[b]], ssems[k]).wait()

        def _ld_dst0():
            pltpu.sync_copy(dst_hbm.at[sid, pl.ds(0, NB0)], dst_v)

        def _ld_dst1():
            pltpu.sync_copy(dst_hbm.at[sid, pl.ds(NB0, NB1)], dst_v.at[pl.ds(0, NB1)])

        pl.when(cid == 0)(_ld_dst0)
        pl.when(cid != 0)(_ld_dst1)
        for chunk in range(nchunk):
            pltpu.sync_copy(zeros_hbm, bufs[0])
            for z in range(RPT // BL):
                pltpu.sync_copy(bufs[0], acc.at[pl.ds(base + z * BL, BL)])
            if RPT % BL:
                pltpu.sync_copy(bufs[0].at[pl.ds(0, RPT % BL)],
                                acc.at[pl.ds(base + (RPT // BL) * BL, RPT % BL)])

            def _ld_src0(chunk=chunk):
                pltpu.sync_copy(src_hbm.at[chunk, sid, pl.ds(0, NB0)], src_v)

            def _ld_src1(chunk=chunk):
                pltpu.sync_copy(src_hbm.at[chunk, sid, pl.ds(NB0, NB1)],
                                src_v.at[pl.ds(0, NB1)])

            pl.when(cid == 0)(_ld_src0)
            pl.when(cid != 0)(_ld_src1)
            plsc.subcore_barrier()

            def outer(i, carry):
                prev = lax.max(i - 1, 0)
                for k in range(NBUF):
                    def _ws(k=k, b=prev * NBUF + k):
                        wait_g(b, k)
                        start_s(b, k)
                    pl.when(i > 0)(_ws)
                for k in range(NBUF):
                    def _dr(k=k, b=prev * NBUF + k):
                        wait_s(b, k)
                    pl.when(i > 0)(_dr)

                    def _sg(k=k, b=i * NBUF + k):
                        start_g(b, k)
                    pl.when(i < nw)(_sg)
                return carry

            lax.fori_loop(0, nw + 1, outer, 0)
            plsc.subcore_barrier()
            pltpu.sync_copy(acc.at[pl.ds(base, RPT)],
                            out_hbm.at[cid, chunk, pl.ds(base, RPT)])
    return _agg


_agg4 = _make_agg(4)
_agg2 = _make_agg(2)


def _dinv(deg_ref):
    return lax.rsqrt(deg_ref[0, :, 0:1] + deg_ref[1, :, 0:1] + 1.0)


def _mm1_body(x_ref, w_ref, deg_ref, out_ref):
    h = jnp.dot(x_ref[...], w_ref[...], preferred_element_type=jnp.float32)
    g = h * _dinv(deg_ref)
    for c in range(out_ref.shape[0]):
        out_ref[c] = g[:, c * 128:(c + 1) * 128]


def _layer_body(aggp_ref, g_ref, deg_ref, b_ref, w_ref, out_ref):
    nin = g_ref.shape[0]
    dinv = _dinv(deg_ref)
    agg = jnp.concatenate([aggp_ref[0, c] + aggp_ref[1, c] for c in range(nin)], axis=1)
    gc = jnp.concatenate([g_ref[c] for c in range(nin)], axis=1)
    t = jnp.maximum(dinv * (agg + gc) + b_ref[...], 0.0)
    h = jnp.dot(t, w_ref[...], preferred_element_type=jnp.float32)
    gn = h * dinv
    for c in range(out_ref.shape[0]):
        out_ref[c] = gn[:, c * 128:(c + 1) * 128]


def _final_body(aggp_ref, g_ref, deg_ref, b_ref, out_ref):
    nin = g_ref.shape[0]
    dinv = _dinv(deg_ref)
    agg = jnp.concatenate([aggp_ref[0, c] + aggp_ref[1, c] for c in range(nin)], axis=1)
    gc = jnp.concatenate([g_ref[c] for c in range(nin)], axis=1)
    out_ref[...] = dinv * (agg + gc) + b_ref[...]


_NROW = NPAD // RBLK


def _mm1(x, w, deg):
    din, dout = w.shape
    return pl.pallas_call(
        _mm1_body,
        grid=(_NROW,),
        in_specs=[
            pl.BlockSpec((RBLK, din), lambda i: (i, 0)),
            pl.BlockSpec((din, dout), lambda i: (0, 0)),
            pl.BlockSpec((2, RBLK, 128), lambda i: (0, i, 0)),
        ],
        out_specs=pl.BlockSpec((dout // 128, RBLK, 128), lambda i: (0, i, 0)),
        out_shape=jax.ShapeDtypeStruct((dout // 128, NPAD, 128), jnp.float32),
    )(x, w, deg)


def _layer(aggp, g, deg, b, w):
    nin = g.shape[0]
    dout = w.shape[1]
    return pl.pallas_call(
        _layer_body,
        grid=(_NROW,),
        in_specs=[
            pl.BlockSpec((2, nin, RBLK, 128), lambda i: (0, 0, i, 0)),
            pl.BlockSpec((nin, RBLK, 128), lambda i: (0, i, 0)),
            pl.BlockSpec((2, RBLK, 128), lambda i: (0, i, 0)),
            pl.BlockSpec((1, nin * 128), lambda i: (0, 0)),
            pl.BlockSpec((nin * 128, dout), lambda i: (0, 0)),
        ],
        out_specs=pl.BlockSpec((dout // 128, RBLK, 128), lambda i: (0, i, 0)),
        out_shape=jax.ShapeDtypeStruct((dout // 128, NPAD, 128), jnp.float32),
    )(aggp, g, deg, b, w)


def _final(aggp, g, deg, b):
    nin = g.shape[0]
    return pl.pallas_call(
        _final_body,
        grid=(_NROW,),
        in_specs=[
            pl.BlockSpec((2, nin, RBLK, 128), lambda i: (0, 0, i, 0)),
            pl.BlockSpec((nin, RBLK, 128), lambda i: (0, i, 0)),
            pl.BlockSpec((2, RBLK, 128), lambda i: (0, i, 0)),
            pl.BlockSpec((1, nin * 128), lambda i: (0, 0)),
        ],
        out_specs=pl.BlockSpec((RBLK, nin * 128), lambda i: (i, 0)),
        out_shape=jax.ShapeDtypeStruct((NPAD, nin * 128), jnp.float32),
    )(aggp, g, deg, b)


def kernel(x, edge_index, W1, b1, W2, b2, W3, b3):
    src = edge_index[0]
    dst = edge_index[1]
    src_pad = jnp.full((EPAD - E,), N, jnp.int32)
    dst_pad = N + (jnp.arange(EPAD - E, dtype=jnp.int32) % (NPAD - N))
    src_p = jnp.concatenate([src, src_pad])
    dst_p = jnp.concatenate([dst, dst_pad])
    dst_padd = N + (jnp.arange(EPADD - E, dtype=jnp.int32) % (NPAD - N))
    dst_r = jnp.concatenate([dst, dst_padd]).reshape(2, 16, NB, DB)
    dst_sl = dst_p.reshape(16, NBT, BL)
    off4 = (jnp.arange(4, dtype=jnp.int32) * NPAD)[:, None]
    src4 = (src_p[None, :] + off4).reshape(4, 16, NBT, BL)
    src2 = (src_p[None, :] + off4[:2]).reshape(2, 16, NBT, BL)
    zeros128 = jnp.zeros((BL, 128), jnp.float32)
    zeros64 = jnp.zeros((64, 128), jnp.float32)
    ones128 = jnp.ones((DB, 128), jnp.float32)
    x_pad = jnp.zeros((NPAD, x.shape[1]), jnp.float32).at[:N].set(x)

    deg = _deg_kernel(dst_r, ones128, zeros64)
    g1 = _mm1(x_pad, W1, deg)
    agg1 = _agg4(g1.reshape(4 * NPAD, 128), src4, dst_sl, zeros128)
    g2 = _layer(agg1, g1, deg, b1.reshape(1, -1), W2)
    agg2 = _agg4(g2.reshape(4 * NPAD, 128), src4, dst_sl, zeros128)
    g3 = _layer(agg2, g2, deg, b2.reshape(1, -1), W3)
    agg3 = _agg2(g3.reshape(2 * NPAD, 128), src2, dst_sl, zeros128)
    z = _final(agg3, g3, deg, b3.reshape(1, -1))
    return z[:N]


# speedup vs baseline: 1.8422x; 1.8422x over previous
import functools

import jax
import jax.numpy as jnp
from jax import lax
from jax.experimental import pallas as pl
from jax.experimental.pallas import tpu as pltpu
from jax.experimental.pallas import tpu_sc as plsc

N = 10000
NPAD = 10240
E = 160000
NB = 40
BL = 128
EPAD = 2 * 16 * NB * BL
RPT = NPAD // 16
RBLK = 1024
NBUF = 2
NBT = 80
NB0 = 56
NB1 = NBT - NB0


def _mesh():
    return plsc.VectorSubcoreMesh(core_axis_name="c", subcore_axis_name="s")


@functools.partial(
    pl.kernel,
    out_type=jax.ShapeDtypeStruct((2, NPAD, 128), jnp.float32),
    mesh=_mesh(),
    name="degk",
    scratch_types=[
        pltpu.VMEM((NB, BL), jnp.int32),
        pltpu.VMEM((BL, 128), jnp.float32),
        pltpu.VMEM((64, 128), jnp.float32),
        pltpu.VMEM_SHARED((NPAD, 128), jnp.float32),
    ],
)
def _deg_kernel(dst_hbm, ones_hbm, zeros_hbm, out_hbm, dst_v, ones_v, zeros_v, acc):
    cid = lax.axis_index("c")
    sid = lax.axis_index("s")
    base = sid * RPT
    pltpu.sync_copy(dst_hbm.at[cid, sid], dst_v)
    pltpu.sync_copy(ones_hbm, ones_v)
    pltpu.sync_copy(zeros_hbm, zeros_v)
    for z in range(RPT // 64):
        pltpu.sync_copy(zeros_v, acc.at[pl.ds(base + z * 64, 64)])
    if RPT % 64:
        pltpu.sync_copy(zeros_v.at[pl.ds(0, RPT % 64)],
                        acc.at[pl.ds(base + (RPT // 64) * 64, RPT % 64)])
    plsc.subcore_barrier()

    def body(b, carry):
        pltpu.sync_copy(ones_v, acc.at[dst_v.at[b]], add=True)
        return carry

    lax.fori_loop(0, NB, body, 0)
    plsc.subcore_barrier()
    pltpu.sync_copy(acc.at[pl.ds(base, RPT)], out_hbm.at[cid, pl.ds(base, RPT)])


def _make_agg(nchunk):
    @functools.partial(
        pl.kernel,
        out_type=jax.ShapeDtypeStruct((2, nchunk, NPAD, 128), jnp.float32),
        mesh=_mesh(),
        name="agg%d" % nchunk,
        scratch_types=[
            pltpu.VMEM((NB0, BL), jnp.int32),
            pltpu.VMEM((NB0, BL), jnp.int32),
            [pltpu.VMEM((BL, 128), jnp.float32)] * NBUF,
            pltpu.VMEM_SHARED((NPAD, 128), jnp.float32),
            [pltpu.SemaphoreType.DMA] * NBUF,
            [pltpu.SemaphoreType.DMA] * NBUF,
        ],
    )
    def _agg(g_hbm, src_hbm, dst_hbm, zeros_hbm, out_hbm,
             src_v, dst_v, bufs, acc, gsems, ssems):
        cid = lax.axis_index("c")
        sid = lax.axis_index("s")
        base = sid * RPT
        nw = lax.select(cid == 0, NB0 // NBUF, NB1 // NBUF)

        def start_g(b, k):
            pltpu.async_copy(g_hbm.at[src_v.at[b]], bufs[k], gsems[k])

        def wait_g(b, k):
            pltpu.make_async_copy(g_hbm.at[src_v.at[b]], bufs[k], gsems[k]).wait()

        def start_s(b, k):
            pltpu.async_copy(bufs[k], acc.at[dst_v.at[b]], ssems[k], add=True)

        def wait_s(b, k):
            pltpu.make_async_copy(bufs[k], acc.at[dst_v.at[b]], ssems[k]).wait()

        def _ld_dst0():
            pltpu.sync_copy(dst_hbm.at[sid, pl.ds(0, NB0)], dst_v)

        def _ld_dst1():
            pltpu.sync_copy(dst_hbm.at[sid, pl.ds(NB0, NB1)], dst_v.at[pl.ds(0, NB1)])

        pl.when(cid == 0)(_ld_dst0)
        pl.when(cid != 0)(_ld_dst1)
        for chunk in range(nchunk):
            pltpu.sync_copy(zeros_hbm, bufs[0])
            for z in range(RPT // BL):
                pltpu.sync_copy(bufs[0], acc.at[pl.ds(base + z * BL, BL)])
            if RPT % BL:
                pltpu.sync_copy(bufs[0].at[pl.ds(0, RPT % BL)],
                                acc.at[pl.ds(base + (RPT // BL) * BL, RPT % BL)])

            def _ld_src0(chunk=chunk):
                pltpu.sync_copy(src_hbm.at[chunk, sid, pl.ds(0, NB0)], src_v)

            def _ld_src1(chunk=chunk):
                pltpu.sync_copy(src_hbm.at[chunk, sid, pl.ds(NB0, NB1)],
                                src_v.at[pl.ds(0, NB1)])

            pl.when(cid == 0)(_ld_src0)
            pl.when(cid != 0)(_ld_src1)
            plsc.subcore_barrier()

            def outer(i, carry):
                prev = lax.max(i - 1, 0)
                for k in range(NBUF):
                    def _ws(k=k, b=prev * NBUF + k):
                        wait_g(b, k)
                        start_s(b, k)
                    pl.when(i > 0)(_ws)
                for k in range(NBUF):
                    def _dr(k=k, b=prev * NBUF + k):
                        wait_s(b, k)
                    pl.when(i > 0)(_dr)

                    def _sg(k=k, b=i * NBUF + k):
                        start_g(b, k)
                    pl.when(i < nw)(_sg)
                return carry

            lax.fori_loop(0, nw + 1, outer, 0)
            plsc.subcore_barrier()
            pltpu.sync_copy(acc.at[pl.ds(base, RPT)],
                            out_hbm.at[cid, chunk, pl.ds(base, RPT)])
    return _agg


_agg4 = _make_agg(4)
_agg2 = _make_agg(2)


def _dinv(deg_ref):
    return lax.rsqrt(deg_ref[0, :, 0:1] + deg_ref[1, :, 0:1] + 1.0)


def _mm1_body(x_ref, w_ref, deg_ref, out_ref):
    h = jnp.dot(x_ref[...], w_ref[...], preferred_element_type=jnp.float32)
    g = h * _dinv(deg_ref)
    for c in range(out_ref.shape[0]):
        out_ref[c] = g[:, c * 128:(c + 1) * 128]


def _layer_body(aggp_ref, g_ref, deg_ref, b_ref, w_ref, out_ref):
    nin = g_ref.shape[0]
    dinv = _dinv(deg_ref)
    agg = jnp.concatenate([aggp_ref[0, c] + aggp_ref[1, c] for c in range(nin)], axis=1)
    gc = jnp.concatenate([g_ref[c] for c in range(nin)], axis=1)
    t = jnp.maximum(dinv * (agg + gc) + b_ref[...], 0.0)
    h = jnp.dot(t, w_ref[...], preferred_element_type=jnp.float32)
    gn = h * dinv
    for c in range(out_ref.shape[0]):
        out_ref[c] = gn[:, c * 128:(c + 1) * 128]


def _final_body(aggp_ref, g_ref, deg_ref, b_ref, out_ref):
    nin = g_ref.shape[0]
    dinv = _dinv(deg_ref)
    agg = jnp.concatenate([aggp_ref[0, c] + aggp_ref[1, c] for c in range(nin)], axis=1)
    gc = jnp.concatenate([g_ref[c] for c in range(nin)], axis=1)
    out_ref[...] = dinv * (agg + gc) + b_ref[...]


_NROW = NPAD // RBLK


def _mm1(x, w, deg):
    din, dout = w.shape
    return pl.pallas_call(
        _mm1_body,
        grid=(_NROW,),
        in_specs=[
            pl.BlockSpec((RBLK, din), lambda i: (i, 0)),
            pl.BlockSpec((din, dout), lambda i: (0, 0)),
            pl.BlockSpec((2, RBLK, 128), lambda i: (0, i, 0)),
        ],
        out_specs=pl.BlockSpec((dout // 128, RBLK, 128), lambda i: (0, i, 0)),
        out_shape=jax.ShapeDtypeStruct((dout // 128, NPAD, 128), jnp.float32),
    )(x, w, deg)


def _layer(aggp, g, deg, b, w):
    nin = g.shape[0]
    dout = w.shape[1]
    return pl.pallas_call(
        _layer_body,
        grid=(_NROW,),
        in_specs=[
            pl.BlockSpec((2, nin, RBLK, 128), lambda i: (0, 0, i, 0)),
            pl.BlockSpec((nin, RBLK, 128), lambda i: (0, i, 0)),
            pl.BlockSpec((2, RBLK, 128), lambda i: (0, i, 0)),
            pl.BlockSpec((1, nin * 128), lambda i: (0, 0)),
            pl.BlockSpec((nin * 128, dout), lambda i: (0, 0)),
        ],
        out_specs=pl.BlockSpec((dout // 128, RBLK, 128), lambda i: (0, i, 0)),
        out_shape=jax.ShapeDtypeStruct((dout // 128, NPAD, 128), jnp.float32),
    )(aggp, g, deg, b, w)


def _final(aggp, g, deg, b):
    nin = g.shape[0]
    return pl.pallas_call(
        _final_body,
        grid=(_NROW,),
        in_specs=[
            pl.BlockSpec((2, nin, RBLK, 128), lambda i: (0, 0, i, 0)),
            pl.BlockSpec((nin, RBLK, 128), lambda i: (0, i, 0)),
            pl.BlockSpec((2, RBLK, 128), lambda i: (0, i, 0)),
            pl.BlockSpec((1, nin * 128), lambda i: (0, 0)),
        ],
        out_specs=pl.BlockSpec((RBLK, nin * 128), lambda i: (i, 0)),
        out_shape=jax.ShapeDtypeStruct((NPAD, nin * 128), jnp.float32),
    )(aggp, g, deg, b)


def kernel(x, edge_index, W1, b1, W2, b2, W3, b3):
    src = edge_index[0]
    dst = edge_index[1]
    src_pad = jnp.full((EPAD - E,), N, jnp.int32)
    dst_pad = N + (jnp.arange(EPAD - E, dtype=jnp.int32) % (NPAD - N))
    src_p = jnp.concatenate([src, src_pad])
    dst_p = jnp.concatenate([dst, dst_pad])
    dst_r = dst_p.reshape(2, 16, NB, BL)
    dst_sl = dst_p.reshape(16, NBT, BL)
    off4 = (jnp.arange(4, dtype=jnp.int32) * NPAD)[:, None]
    src4 = (src_p[None, :] + off4).reshape(4, 16, NBT, BL)
    src2 = (src_p[None, :] + off4[:2]).reshape(2, 16, NBT, BL)
    zeros128 = jnp.zeros((BL, 128), jnp.float32)
    zeros64 = jnp.zeros((64, 128), jnp.float32)
    ones128 = jnp.ones((BL, 128), jnp.float32)
    x_pad = jnp.zeros((NPAD, x.shape[1]), jnp.float32).at[:N].set(x)

    deg = _deg_kernel(dst_r, ones128, zeros64)
    g1 = _mm1(x_pad, W1, deg)
    agg1 = _agg4(g1.reshape(4 * NPAD, 128), src4, dst_sl, zeros128)
    g2 = _layer(agg1, g1, deg, b1.reshape(1, -1), W2)
    agg2 = _agg4(g2.reshape(4 * NPAD, 128), src4, dst_sl, zeros128)
    g3 = _layer(agg2, g2, deg, b2.reshape(1, -1), W3)
    agg3 = _agg2(g3.reshape(2 * NPAD, 128), src2, dst_sl, zeros128)
    z = _final(agg3, g3, deg, b3.reshape(1, -1))
    return z[:N]
